# baseline (speedup 1.0000x reference)
import functools

import jax
import jax.numpy as jnp
from jax import lax
from jax.experimental import pallas as pl
from jax.experimental.pallas import tpu as pltpu
from jax.experimental.pallas import tpu_sc as plsc

_NUM_CORES = 2
_NUM_SUBCORES = 16
_NUM_WORKERS = _NUM_CORES * _NUM_SUBCORES

_CHUNK = 512


@functools.lru_cache(maxsize=None)
def _build_gather(n_idx: int, vocab: int, dim: int):
    assert n_idx % (_NUM_WORKERS * _CHUNK) == 0
    per_worker = n_idx // _NUM_WORKERS
    n_chunks = per_worker // _CHUNK

    mesh = plsc.VectorSubcoreMesh(core_axis_name="c", subcore_axis_name="s")

    @functools.partial(
        pl.kernel,
        mesh=mesh,
        out_type=jax.ShapeDtypeStruct((n_idx, dim), jnp.float32),
        scratch_types=[
            pltpu.VMEM((_CHUNK,), jnp.int32),
            pltpu.VMEM((_CHUNK, dim), jnp.float32),
            pltpu.SemaphoreType.DMA,
        ],
        compiler_params=pltpu.CompilerParams(use_tc_tiling_on_sc=False),
    )
    def gather_kernel(tok_hbm, w_hbm, out_hbm, idx_v, rows_v, sem):
        wid = lax.axis_index("s") * _NUM_CORES + lax.axis_index("c")
        wbase = wid * per_worker

        def step(g, carry):
            base = wbase + g * _CHUNK
            pltpu.sync_copy(tok_hbm.at[pl.ds(base, _CHUNK)], idx_v)
            pltpu.async_copy(w_hbm.at[idx_v], rows_v, sem).wait()
            pltpu.sync_copy(rows_v, out_hbm.at[pl.ds(base, _CHUNK)])
            return carry

        lax.fori_loop(0, n_chunks, step, 0)

    return gather_kernel


def kernel(token_ids, weight):
    batch, hist = token_ids.shape
    vocab, dim = weight.shape
    flat = token_ids.reshape(-1).astype(jnp.int32)
    out = _build_gather(flat.shape[0], vocab, dim)(flat, weight)
    return out.reshape(batch, hist, dim)


# speedup vs baseline: 1.0381x; 1.0381x over previous
import functools

import jax
import jax.numpy as jnp
from jax import lax
from jax.experimental import pallas as pl
from jax.experimental.pallas import tpu as pltpu
from jax.experimental.pallas import tpu_sc as plsc

_NUM_CORES = 2
_NUM_SUBCORES = 16
_NUM_WORKERS = _NUM_CORES * _NUM_SUBCORES

_CHUNK = 512
_NBUF = 2


@functools.lru_cache(maxsize=None)
def _build_gather(n_idx: int, vocab: int, dim: int):
    assert n_idx % (_NUM_WORKERS * _CHUNK * _NBUF) == 0
    per_worker = n_idx // _NUM_WORKERS
    n_groups = per_worker // (_CHUNK * _NBUF)

    mesh = plsc.VectorSubcoreMesh(core_axis_name="c", subcore_axis_name="s")

    @functools.partial(
        pl.kernel,
        mesh=mesh,
        out_type=jax.ShapeDtypeStruct((n_idx, dim), jnp.float32),
        scratch_types=(
            [pltpu.VMEM((_CHUNK,), jnp.int32) for _ in range(_NBUF)]
            + [pltpu.VMEM((_CHUNK, dim), jnp.float32) for _ in range(_NBUF)]
            + [pltpu.SemaphoreType.DMA for _ in range(3 * _NBUF)]
        ),
        compiler_params=pltpu.CompilerParams(use_tc_tiling_on_sc=False),
    )
    def gather_kernel(tok_hbm, w_hbm, out_hbm, *scratch):
        idx_bufs = scratch[:_NBUF]
        rows_bufs = scratch[_NBUF : 2 * _NBUF]
        sem_i = scratch[2 * _NBUF : 3 * _NBUF]
        sem_g = scratch[3 * _NBUF : 4 * _NBUF]
        sem_o = scratch[4 * _NBUF : 5 * _NBUF]

        wid = lax.axis_index("s") * _NUM_CORES + lax.axis_index("c")
        wbase = wid * per_worker

        def chunk_base(t, b):
            return wbase + (t * _NBUF + b) * _CHUNK

        for b in range(_NBUF):
            pltpu.async_copy(
                tok_hbm.at[pl.ds(chunk_base(0, b), _CHUNK)], idx_bufs[b], sem_i[b]
            )

        def group(t, carry):
            for b in range(_NBUF):
                base = chunk_base(t, b)

                @pl.when(t > 0)
                def _():
                    pltpu.make_async_copy(
                        rows_bufs[b], out_hbm.at[pl.ds(base, _CHUNK)], sem_o[b]
                    ).wait()

                pltpu.make_async_copy(
                    tok_hbm.at[pl.ds(base, _CHUNK)], idx_bufs[b], sem_i[b]
                ).wait()
                pltpu.async_copy(w_hbm.at[idx_bufs[b]], rows_bufs[b], sem_g[b])

            for b in range(_NBUF):
                base = chunk_base(t, b)
                pltpu.make_async_copy(
                    w_hbm.at[idx_bufs[b]], rows_bufs[b], sem_g[b]
                ).wait()
                pltpu.async_copy(
                    rows_bufs[b], out_hbm.at[pl.ds(base, _CHUNK)], sem_o[b]
                )

                @pl.when(t + 1 < n_groups)
                def _():
                    pltpu.async_copy(
                        tok_hbm.at[pl.ds(chunk_base(t + 1, b), _CHUNK)],
                        idx_bufs[b],
                        sem_i[b],
                    )

            return carry

        lax.fori_loop(0, n_groups, group, 0)

        for b in range(_NBUF):
            base = chunk_base(n_groups - 1, b)
            pltpu.make_async_copy(
                rows_bufs[b], out_hbm.at[pl.ds(base, _CHUNK)], sem_o[b]
            ).wait()

    return gather_kernel


def kernel(token_ids, weight):
    batch, hist = token_ids.shape
    vocab, dim = weight.shape
    flat = token_ids.reshape(-1).astype(jnp.int32)
    out = _build_gather(flat.shape[0], vocab, dim)(flat, weight)
    return out.reshape(batch, hist, dim)
